# Initial kernel scaffold; baseline (speedup 1.0000x reference)
import jax
import jax.numpy as jnp
from jax.experimental import pallas as pl


def kernel(node_features, edge_index, Wq, bq, Wk, bk, Wv, bv, Wo, bo, ln_g, ln_b, gpW1, gpb1, gpW2, gpb2):
    raise NotImplementedError("write your pallas kernel here")


# speedup vs baseline: 126.7146x; 126.7146x over previous
import functools

import jax
import jax.numpy as jnp
import numpy as np
from jax import lax
from jax.experimental import pallas as pl
from jax.experimental.pallas import tpu as pltpu
from jax.experimental.pallas import tpu_sc as plsc

_D = 128
_H = 8
_DH = 16
_L = 4
_NC = 2
_NS = 16
_HC = _H // _NC
_DC = _D // _NC
_EB = 128
_INV_SQRT_DH = np.float32(1.0 / np.sqrt(_DH))


def _fori(lo, hi, body):
    lax.fori_loop(jnp.int32(lo), jnp.int32(hi),
                  lambda i, c: (body(i), c)[1], jnp.int32(0))


def _qkv_body(x_ref, wq_ref, bq_ref, wk_ref, bk_ref, wv_ref, bv_ref,
              q_ref, k_ref, v_ref):
    x = x_ref[...]
    q = jnp.dot(x, wq_ref[...], preferred_element_type=jnp.float32) + bq_ref[...]
    k = jnp.dot(x, wk_ref[...], preferred_element_type=jnp.float32) + bk_ref[...]
    v = jnp.dot(x, wv_ref[...], preferred_element_type=jnp.float32) + bv_ref[...]
    q_ref[0] = q[:, :_DC]
    q_ref[1] = q[:, _DC:]
    k_ref[0] = k[:, :_DC]
    k_ref[1] = k[:, _DC:]
    v_ref[0] = v[:, :_DC]
    v_ref[1] = v[:, _DC:]


def _post_body(att_ref, den_ref, x_ref, wo_ref, bo_ref, g_ref, b_ref, sel_ref,
               o_ref):
    att = jnp.concatenate([att_ref[0], att_ref[1]], axis=1)
    den = jnp.concatenate([den_ref[0][:, :_HC], den_ref[1][:, :_HC]], axis=1)
    x = x_ref[...]
    inv = np.float32(1.0) / (den + np.float32(1e-30))
    invb = jnp.dot(inv, sel_ref[...], preferred_element_type=jnp.float32)
    out = jnp.dot(att * invb, wo_ref[...], preferred_element_type=jnp.float32)
    out = out + bo_ref[...]
    deg = den_ref[0][:, _H:_H + 1]
    h = jnp.where(deg > 0, out, x)
    y = h + x
    mu = jnp.mean(y, axis=1, keepdims=True)
    var = jnp.mean((y - mu) ** 2, axis=1, keepdims=True)
    o_ref[...] = (y - mu) * lax.rsqrt(var + np.float32(1e-5)) * g_ref[...] + b_ref[...]


def _pool_body(x_ref, w1_ref, b1_ref, w2_ref, b2_ref, o_ref):
    n = x_ref.shape[0]
    s = jnp.sum(x_ref[...], axis=0, keepdims=True) * np.float32(1.0 / n)
    g = jnp.dot(s, w1_ref[...], preferred_element_type=jnp.float32) + b1_ref[...]
    g = jnp.maximum(g, np.float32(0.0))
    o_ref[...] = jnp.dot(g, w2_ref[...], preferred_element_type=jnp.float32) + b2_ref[...]


def _edge_body(nblk, per_tile, rpt,
               q_hbm, k_hbm, v_hbm, src_hbm, dst_hbm, dstw_hbm,
               att_out, den_out,
               src_v, dst_v, dstw_v, q_rows, k_rows, v_rows, wv_v, ex_v, sem,
               att_sh, den_sh):
    cid = lax.axis_index("c")
    sid = lax.axis_index("s")
    row0 = sid * rpt

    zeros16 = jnp.zeros((16,), jnp.float32)

    def _zero_row(i):
        for h in range(_HC):
            wv_v[i, pl.ds(h * _DH, _DH)] = zeros16
        ex_v[i, :] = zeros16
    _fori(0, _EB, _zero_row)

    n_full = rpt // _EB
    rem = rpt - n_full * _EB

    def _zero_sh(j):
        pltpu.sync_copy(wv_v, att_sh.at[pl.ds(row0 + j * _EB, _EB)])
        pltpu.sync_copy(ex_v, den_sh.at[pl.ds(row0 + j * _EB, _EB)])
    _fori(0, n_full, _zero_sh)
    if rem:
        pltpu.sync_copy(wv_v.at[pl.ds(0, rem)],
                        att_sh.at[pl.ds(row0 + n_full * _EB, rem)])
        pltpu.sync_copy(ex_v.at[pl.ds(0, rem)],
                        den_sh.at[pl.ds(row0 + n_full * _EB, rem)])
    plsc.subcore_barrier()

    lane = lax.iota(jnp.int32, 16)
    ex_init = jnp.where(lane == _H, np.float32(1.0), np.float32(0.0))
    masks = [lane == h for h in range(_HC)]
    tbase = sid * per_tile

    def _blk(b):
        base = tbase + b * _EB
        pltpu.sync_copy(src_hbm.at[pl.ds(base, _EB)], src_v)
        pltpu.sync_copy(dst_hbm.at[pl.ds(base, _EB)], dst_v)
        pltpu.sync_copy(dstw_hbm.at[pl.ds(base, _EB)], dstw_v)
        cq = pltpu.async_copy(q_hbm.at[cid].at[dst_v], q_rows, sem)
        ck = pltpu.async_copy(k_hbm.at[cid].at[src_v], k_rows, sem)
        cv = pltpu.async_copy(v_hbm.at[cid].at[src_v], v_rows, sem)
        cq.wait()
        ck.wait()
        cv.wait()

        def _edge(e):
            exp_pack = ex_init
            for h in range(_HC):
                sl = pl.ds(h * _DH, _DH)
                qh = q_rows[e, sl]
                kh = k_rows[e, sl]
                s = jnp.sum(qh * kh) * _INV_SQRT_DH
                eh = jnp.exp(jnp.broadcast_to(s, (16,)))
                wv_v[e, sl] = eh * v_rows[e, sl]
                exp_pack = jnp.where(masks[h], eh, exp_pack)
            ex_v[e, :] = exp_pack
        _fori(0, _EB, _edge)

        pltpu.sync_copy(ex_v, den_sh.at[dstw_v], add=True)
        pltpu.sync_copy(wv_v, att_sh.at[dstw_v], add=True)
    _fori(0, nblk, _blk)
    plsc.subcore_barrier()

    pltpu.sync_copy(att_sh.at[pl.ds(row0, rpt)], att_out.at[cid, pl.ds(row0, rpt)])
    pltpu.sync_copy(den_sh.at[pl.ds(row0, rpt)], den_out.at[cid, pl.ds(row0, rpt)])


def kernel(node_features, edge_index, Wq, bq, Wk, bk, Wv, bv, Wo, bo,
           ln_g, ln_b, gpW1, gpb1, gpW2, gpb2):
    n = node_features.shape[0]
    e = edge_index.shape[1]
    e2 = 2 * e
    f32 = jnp.float32

    ei32 = edge_index.astype(jnp.int32)
    src = jnp.concatenate([ei32[1], ei32[0]])
    dst = jnp.concatenate([ei32[0], ei32[1]])
    per_tile = -(-e2 // (_NS * _EB)) * _EB
    e_pad = per_tile * _NS
    pad = e_pad - e2
    if pad:
        z = jnp.zeros((pad,), jnp.int32)
        src = jnp.concatenate([src, z])
        dst = jnp.concatenate([dst, z])
    dstw = jnp.where(src == dst, jnp.int32(n), dst)

    nblk = per_tile // _EB
    r_rows = -(-(n + 1) // 128) * 128
    rpt = r_rows // _NS

    wq_t = jnp.transpose(Wq, (0, 2, 1)).astype(f32)
    wk_t = jnp.transpose(Wk, (0, 2, 1)).astype(f32)
    wv_t = jnp.transpose(Wv, (0, 2, 1)).astype(f32)
    wo_t = jnp.transpose(Wo, (0, 2, 1)).astype(f32)
    sel = jnp.repeat(jnp.eye(_H, dtype=f32), _DH, axis=1)

    bn = 1000
    grid_n = n // bn

    qkv_call = pl.pallas_call(
        _qkv_body,
        grid=(grid_n,),
        in_specs=[
            pl.BlockSpec((bn, _D), lambda i: (i, i * 0)),
            pl.BlockSpec((_D, _D), lambda i: (i * 0, i * 0)),
            pl.BlockSpec((1, _D), lambda i: (i * 0, i * 0)),
            pl.BlockSpec((_D, _D), lambda i: (i * 0, i * 0)),
            pl.BlockSpec((1, _D), lambda i: (i * 0, i * 0)),
            pl.BlockSpec((_D, _D), lambda i: (i * 0, i * 0)),
            pl.BlockSpec((1, _D), lambda i: (i * 0, i * 0)),
        ],
        out_specs=[
            pl.BlockSpec((_NC, bn, _DC), lambda i: (i * 0, i, i * 0)),
            pl.BlockSpec((_NC, bn, _DC), lambda i: (i * 0, i, i * 0)),
            pl.BlockSpec((_NC, bn, _DC), lambda i: (i * 0, i, i * 0)),
        ],
        out_shape=[jax.ShapeDtypeStruct((_NC, n, _DC), f32)] * 3,
    )

    edge_call = pl.kernel(
        functools.partial(_edge_body, nblk, per_tile, rpt),
        out_type=(jax.ShapeDtypeStruct((_NC, r_rows, _DC), f32),
                  jax.ShapeDtypeStruct((_NC, r_rows, 16), f32)),
        mesh=plsc.VectorSubcoreMesh(core_axis_name="c", subcore_axis_name="s"),
        compiler_params=pltpu.CompilerParams(needs_layout_passes=False,
                                             use_tc_tiling_on_sc=False),
        scratch_types=[
            pltpu.VMEM((_EB,), jnp.int32),
            pltpu.VMEM((_EB,), jnp.int32),
            pltpu.VMEM((_EB,), jnp.int32),
            pltpu.VMEM((_EB, _DC), f32),
            pltpu.VMEM((_EB, _DC), f32),
            pltpu.VMEM((_EB, _DC), f32),
            pltpu.VMEM((_EB, _DC), f32),
            pltpu.VMEM((_EB, 16), f32),
            pltpu.SemaphoreType.DMA,
            pltpu.VMEM_SHARED((r_rows, _DC), f32),
            pltpu.VMEM_SHARED((r_rows, 16), f32),
        ],
    )

    post_call = pl.pallas_call(
        _post_body,
        grid=(grid_n,),
        in_specs=[
            pl.BlockSpec((_NC, bn, _DC), lambda i: (i * 0, i, i * 0)),
            pl.BlockSpec((_NC, bn, 16), lambda i: (i * 0, i, i * 0)),
            pl.BlockSpec((bn, _D), lambda i: (i, i * 0)),
            pl.BlockSpec((_D, _D), lambda i: (i * 0, i * 0)),
            pl.BlockSpec((1, _D), lambda i: (i * 0, i * 0)),
            pl.BlockSpec((1, _D), lambda i: (i * 0, i * 0)),
            pl.BlockSpec((1, _D), lambda i: (i * 0, i * 0)),
            pl.BlockSpec((_H, _D), lambda i: (i * 0, i * 0)),
        ],
        out_specs=pl.BlockSpec((bn, _D), lambda i: (i, i * 0)),
        out_shape=jax.ShapeDtypeStruct((n, _D), f32),
    )

    pool_call = pl.pallas_call(
        _pool_body,
        out_shape=jax.ShapeDtypeStruct((1, _D), f32),
    )

    x = node_features.astype(f32)
    for l in range(_L):
        q, k, v = qkv_call(x, wq_t[l], bq[l].reshape(1, _D),
                           wk_t[l], bk[l].reshape(1, _D),
                           wv_t[l], bv[l].reshape(1, _D))
        att_p, den_p = edge_call(q, k, v, src, dst, dstw)
        x = post_call(att_p, den_p, x, wo_t[l], bo[l].reshape(1, _D),
                      ln_g[l].reshape(1, _D), ln_b[l].reshape(1, _D), sel)

    emb = pool_call(x, jnp.transpose(gpW1).astype(f32), gpb1.reshape(1, _D),
                    jnp.transpose(gpW2).astype(f32), gpb2.reshape(1, _D))
    return x.astype(jnp.float64), emb.reshape(_D).astype(jnp.float64)


# speedup vs baseline: 222.0082x; 1.7520x over previous
import functools

import jax
import jax.numpy as jnp
import numpy as np
from jax import lax
from jax.experimental import pallas as pl
from jax.experimental.pallas import tpu as pltpu
from jax.experimental.pallas import tpu_sc as plsc

_D = 128
_H = 8
_DH = 16
_L = 4
_NC = 2
_NS = 16
_HC = _H // _NC
_DC = _D // _NC
_EB = 128
_INV_SQRT_DH = np.float32(1.0 / np.sqrt(_DH))


def _fori(lo, hi, body):
    lax.fori_loop(jnp.int32(lo), jnp.int32(hi),
                  lambda i, c: (body(i), c)[1], jnp.int32(0))


def _qkv_body(x_ref, wq_ref, bq_ref, wk_ref, bk_ref, wv_ref, bv_ref,
              q_ref, k_ref, v_ref):
    x = x_ref[...]
    q = jnp.dot(x, wq_ref[...], preferred_element_type=jnp.float32) + bq_ref[...]
    k = jnp.dot(x, wk_ref[...], preferred_element_type=jnp.float32) + bk_ref[...]
    v = jnp.dot(x, wv_ref[...], preferred_element_type=jnp.float32) + bv_ref[...]
    q_ref[0] = q[:, :_DC]
    q_ref[1] = q[:, _DC:]
    k_ref[0] = k[:, :_DC]
    k_ref[1] = k[:, _DC:]
    v_ref[0] = v[:, :_DC]
    v_ref[1] = v[:, _DC:]


def _post_body(att_ref, den_ref, x_ref, wo_ref, bo_ref, g_ref, b_ref, sel_ref,
               o_ref):
    att = jnp.concatenate([att_ref[0], att_ref[1]], axis=1)
    den = jnp.concatenate([den_ref[0][:, :_HC], den_ref[1][:, :_HC]], axis=1)
    x = x_ref[...]
    inv = np.float32(1.0) / (den + np.float32(1e-30))
    invb = jnp.dot(inv, sel_ref[...], preferred_element_type=jnp.float32)
    out = jnp.dot(att * invb, wo_ref[...], preferred_element_type=jnp.float32)
    out = out + bo_ref[...]
    deg = den_ref[0][:, _H:_H + 1]
    h = jnp.where(deg > 0, out, x)
    y = h + x
    mu = jnp.mean(y, axis=1, keepdims=True)
    var = jnp.mean((y - mu) ** 2, axis=1, keepdims=True)
    o_ref[...] = (y - mu) * lax.rsqrt(var + np.float32(1e-5)) * g_ref[...] + b_ref[...]


def _pool_body(x_ref, w1_ref, b1_ref, w2_ref, b2_ref, o_ref):
    n = x_ref.shape[0]
    s = jnp.sum(x_ref[...], axis=0, keepdims=True) * np.float32(1.0 / n)
    g = jnp.dot(s, w1_ref[...], preferred_element_type=jnp.float32) + b1_ref[...]
    g = jnp.maximum(g, np.float32(0.0))
    o_ref[...] = jnp.dot(g, w2_ref[...], preferred_element_type=jnp.float32) + b2_ref[...]


def _edge_body(nblk, per_tile, rpt,
               q_hbm, k_hbm, v_hbm, src_hbm, dst_hbm, dstw_hbm,
               att_out, den_out,
               src_v, dst_v, dstw_v, q_rows, k_rows, v_rows, wv_v, ex_v, sem,
               att_sh, den_sh):
    cid = lax.axis_index("c")
    sid = lax.axis_index("s")
    row0 = sid * rpt

    zeros16 = jnp.zeros((16,), jnp.float32)

    def _zero_row(i):
        for h in range(_HC):
            wv_v[i, pl.ds(h * _DH, _DH)] = zeros16
        ex_v[i, :] = zeros16
    _fori(0, _EB, _zero_row)

    n_full = rpt // _EB
    rem = rpt - n_full * _EB

    def _zero_sh(j):
        pltpu.sync_copy(wv_v, att_sh.at[pl.ds(row0 + j * _EB, _EB)])
        pltpu.sync_copy(ex_v, den_sh.at[pl.ds(row0 + j * _EB, _EB)])
    _fori(0, n_full, _zero_sh)
    if rem:
        pltpu.sync_copy(wv_v.at[pl.ds(0, rem)],
                        att_sh.at[pl.ds(row0 + n_full * _EB, rem)])
        pltpu.sync_copy(ex_v.at[pl.ds(0, rem)],
                        den_sh.at[pl.ds(row0 + n_full * _EB, rem)])
    plsc.subcore_barrier()

    lane = lax.iota(jnp.int32, 16)
    ex_init = jnp.where(lane == _H, np.float32(1.0), np.float32(0.0))
    masks = [lane == h for h in range(_HC)]
    tbase = sid * per_tile

    def _blk(b):
        base = tbase + b * _EB
        pltpu.sync_copy(src_hbm.at[pl.ds(base, _EB)], src_v)
        pltpu.sync_copy(dst_hbm.at[pl.ds(base, _EB)], dst_v)
        pltpu.sync_copy(dstw_hbm.at[pl.ds(base, _EB)], dstw_v)
        cq = pltpu.async_copy(q_hbm.at[cid].at[dst_v], q_rows, sem)
        ck = pltpu.async_copy(k_hbm.at[cid].at[src_v], k_rows, sem)
        cv = pltpu.async_copy(v_hbm.at[cid].at[src_v], v_rows, sem)
        cq.wait()
        ck.wait()
        cv.wait()

        @plsc.parallel_loop(jnp.int32(0), jnp.int32(_EB), jnp.int32(1), unroll=4)
        def _edge(e):
            exp_pack = ex_init
            for h in range(_HC):
                sl = pl.ds(h * _DH, _DH)
                qh = q_rows[e, sl]
                kh = k_rows[e, sl]
                s = jnp.sum(qh * kh) * _INV_SQRT_DH
                eh = jnp.exp(jnp.broadcast_to(s, (16,)))
                wv_v[e, sl] = eh * v_rows[e, sl]
                exp_pack = jnp.where(masks[h], eh, exp_pack)
            ex_v[e, :] = exp_pack

        pltpu.sync_copy(ex_v, den_sh.at[dstw_v], add=True)
        pltpu.sync_copy(wv_v, att_sh.at[dstw_v], add=True)
    _fori(0, nblk, _blk)
    plsc.subcore_barrier()

    pltpu.sync_copy(att_sh.at[pl.ds(row0, rpt)], att_out.at[cid, pl.ds(row0, rpt)])
    pltpu.sync_copy(den_sh.at[pl.ds(row0, rpt)], den_out.at[cid, pl.ds(row0, rpt)])


def kernel(node_features, edge_index, Wq, bq, Wk, bk, Wv, bv, Wo, bo,
           ln_g, ln_b, gpW1, gpb1, gpW2, gpb2):
    n = node_features.shape[0]
    e = edge_index.shape[1]
    e2 = 2 * e
    f32 = jnp.float32

    ei32 = edge_index.astype(jnp.int32)
    src = jnp.concatenate([ei32[1], ei32[0]])
    dst = jnp.concatenate([ei32[0], ei32[1]])
    per_tile = -(-e2 // (_NS * _EB)) * _EB
    e_pad = per_tile * _NS
    pad = e_pad - e2
    if pad:
        z = jnp.zeros((pad,), jnp.int32)
        src = jnp.concatenate([src, z])
        dst = jnp.concatenate([dst, z])
    dstw = jnp.where(src == dst, jnp.int32(n), dst)

    nblk = per_tile // _EB
    r_rows = -(-(n + 1) // 128) * 128
    rpt = r_rows // _NS

    wq_t = jnp.transpose(Wq, (0, 2, 1)).astype(f32)
    wk_t = jnp.transpose(Wk, (0, 2, 1)).astype(f32)
    wv_t = jnp.transpose(Wv, (0, 2, 1)).astype(f32)
    wo_t = jnp.transpose(Wo, (0, 2, 1)).astype(f32)
    sel = jnp.repeat(jnp.eye(_H, dtype=f32), _DH, axis=1)

    bn = 1000
    grid_n = n // bn

    qkv_call = pl.pallas_call(
        _qkv_body,
        grid=(grid_n,),
        in_specs=[
            pl.BlockSpec((bn, _D), lambda i: (i, i * 0)),
            pl.BlockSpec((_D, _D), lambda i: (i * 0, i * 0)),
            pl.BlockSpec((1, _D), lambda i: (i * 0, i * 0)),
            pl.BlockSpec((_D, _D), lambda i: (i * 0, i * 0)),
            pl.BlockSpec((1, _D), lambda i: (i * 0, i * 0)),
            pl.BlockSpec((_D, _D), lambda i: (i * 0, i * 0)),
            pl.BlockSpec((1, _D), lambda i: (i * 0, i * 0)),
        ],
        out_specs=[
            pl.BlockSpec((_NC, bn, _DC), lambda i: (i * 0, i, i * 0)),
            pl.BlockSpec((_NC, bn, _DC), lambda i: (i * 0, i, i * 0)),
            pl.BlockSpec((_NC, bn, _DC), lambda i: (i * 0, i, i * 0)),
        ],
        out_shape=[jax.ShapeDtypeStruct((_NC, n, _DC), f32)] * 3,
    )

    edge_call = pl.kernel(
        functools.partial(_edge_body, nblk, per_tile, rpt),
        out_type=(jax.ShapeDtypeStruct((_NC, r_rows, _DC), f32),
                  jax.ShapeDtypeStruct((_NC, r_rows, 16), f32)),
        mesh=plsc.VectorSubcoreMesh(core_axis_name="c", subcore_axis_name="s"),
        compiler_params=pltpu.CompilerParams(needs_layout_passes=False,
                                             use_tc_tiling_on_sc=False),
        scratch_types=[
            pltpu.VMEM((_EB,), jnp.int32),
            pltpu.VMEM((_EB,), jnp.int32),
            pltpu.VMEM((_EB,), jnp.int32),
            pltpu.VMEM((_EB, _DC), f32),
            pltpu.VMEM((_EB, _DC), f32),
            pltpu.VMEM((_EB, _DC), f32),
            pltpu.VMEM((_EB, _DC), f32),
            pltpu.VMEM((_EB, 16), f32),
            pltpu.SemaphoreType.DMA,
            pltpu.VMEM_SHARED((r_rows, _DC), f32),
            pltpu.VMEM_SHARED((r_rows, 16), f32),
        ],
    )

    post_call = pl.pallas_call(
        _post_body,
        grid=(grid_n,),
        in_specs=[
            pl.BlockSpec((_NC, bn, _DC), lambda i: (i * 0, i, i * 0)),
            pl.BlockSpec((_NC, bn, 16), lambda i: (i * 0, i, i * 0)),
            pl.BlockSpec((bn, _D), lambda i: (i, i * 0)),
            pl.BlockSpec((_D, _D), lambda i: (i * 0, i * 0)),
            pl.BlockSpec((1, _D), lambda i: (i * 0, i * 0)),
            pl.BlockSpec((1, _D), lambda i: (i * 0, i * 0)),
            pl.BlockSpec((1, _D), lambda i: (i * 0, i * 0)),
            pl.BlockSpec((_H, _D), lambda i: (i * 0, i * 0)),
        ],
        out_specs=pl.BlockSpec((bn, _D), lambda i: (i, i * 0)),
        out_shape=jax.ShapeDtypeStruct((n, _D), f32),
    )

    pool_call = pl.pallas_call(
        _pool_body,
        out_shape=jax.ShapeDtypeStruct((1, _D), f32),
    )

    x = node_features.astype(f32)
    for l in range(_L):
        q, k, v = qkv_call(x, wq_t[l], bq[l].reshape(1, _D),
                           wk_t[l], bk[l].reshape(1, _D),
                           wv_t[l], bv[l].reshape(1, _D))
        att_p, den_p = edge_call(q, k, v, src, dst, dstw)
        x = post_call(att_p, den_p, x, wo_t[l], bo[l].reshape(1, _D),
                      ln_g[l].reshape(1, _D), ln_b[l].reshape(1, _D), sel)

    emb = pool_call(x, jnp.transpose(gpW1).astype(f32), gpb1.reshape(1, _D),
                    jnp.transpose(gpW2).astype(f32), gpb2.reshape(1, _D))
    return x.astype(jnp.float64), emb.reshape(_D).astype(jnp.float64)
